# baseline (speedup 1.0000x reference)
import functools

import jax
import jax.numpy as jnp
from jax import lax
from jax.experimental import pallas as pl
from jax.experimental.pallas import tpu as pltpu
from jax.experimental.pallas import tpu_sc as plsc

B, N, D = 16, 4096, 256
NC, NS, L = 2, 16, 16
NJ = D // L
BS = 2
BPC = BS // NC
WPB = NS // BPC
RW = N // WPB
CH = 128
NCHUNK = RW // CH
RSTEP = 8

_mesh = plsc.VectorSubcoreMesh(core_axis_name="c", subcore_axis_name="s")


def _tree(op, xs):
    while len(xs) > 1:
        xs = [op(xs[i], xs[i + 1]) for i in range(0, len(xs) - 1, 2)] + (
            [xs[-1]] if len(xs) % 2 else [])
    return xs[0]


def _sqrt_sc(v):
    s = jnp.full((L,), 1.0, jnp.float32)
    for t in range(1, 17):
        s = jnp.where(v >= jnp.float32(4.0 ** t), s * 2.0, s)
        s = jnp.where(v < jnp.float32(4.0 ** (1 - t)), s * 0.5, s)
    for _ in range(4):
        s = jnp.float32(0.5) * (s + v / s)
    return jnp.where(v > 0.0, s, 0.0)


@functools.partial(
    pl.kernel,
    mesh=_mesh,
    out_type=jax.ShapeDtypeStruct((BS, 4, D), jnp.float32),
    scratch_types=[
        pltpu.VMEM((CH * D,), jnp.float32),
        pltpu.VMEM((CH * D,), jnp.float32),
        pltpu.VMEM((4, D), jnp.float32),
        pltpu.VMEM((4, D), jnp.float32),
        pltpu.VMEM_SHARED((NS, 4, D), jnp.float32),
        pltpu.SemaphoreType.DMA,
        pltpu.SemaphoreType.DMA,
    ],
)
def _pool_sc(x_hbm, out_hbm, buf0, buf1, acc, pacc, shared, sem0, sem1):
    c = lax.axis_index("c")
    s = lax.axis_index("s")
    b = c * BPC + s // WPB
    rank = s % WPB
    r0 = rank * RW

    def ibody(j, carry):
        sl = pl.ds(j * L, L)
        acc[0, sl] = jnp.full((L,), -jnp.inf, jnp.float32)
        acc[1, sl] = jnp.full((L,), jnp.inf, jnp.float32)
        acc[2, sl] = jnp.zeros((L,), jnp.float32)
        acc[3, sl] = jnp.zeros((L,), jnp.float32)
        return carry

    lax.fori_loop(0, NJ, ibody, 0)

    def copy(k, buf, sem):
        return pltpu.make_async_copy(
            x_hbm.at[b, pl.ds((r0 + k * CH) * D, CH * D)], buf, sem)

    copy(0, buf0, sem0).start()
    copy(1, buf1, sem1).start()

    def process(buf):
        def jbody(j, carry):
            sl = pl.ds(j * L, L)
            base = j * L

            @plsc.parallel_loop(
                base, base + CH * D, step=RSTEP * D,
                carry=(acc[0, sl], acc[1, sl], acc[2, sl], acc[3, sl]))
            def rbody(q, t):
                mx, mn, sm, ss = t
                v = [buf[pl.ds(q + u * D, L)] for u in range(RSTEP)]
                mx = jnp.maximum(mx, _tree(jnp.maximum, v))
                mn = jnp.minimum(mn, _tree(jnp.minimum, v))
                sm = sm + _tree(lax.add, v)
                ss = ss + _tree(lax.add, [vi * vi for vi in v])
                return (mx, mn, sm, ss)

            mx, mn, sm, ss = rbody
            acc[0, sl] = mx
            acc[1, sl] = mn
            acc[2, sl] = sm
            acc[3, sl] = ss
            return carry

        lax.fori_loop(0, NJ, jbody, 0)

    def outer(i, carry):
        kk = i * 2
        copy(kk, buf0, sem0).wait()
        process(buf0)

        @pl.when(kk + 2 < NCHUNK)
        def _():
            copy(kk + 2, buf0, sem0).start()

        copy(kk + 1, buf1, sem1).wait()
        process(buf1)

        @pl.when(kk + 3 < NCHUNK)
        def _():
            copy(kk + 3, buf1, sem1).start()

        return carry

    lax.fori_loop(0, NCHUNK // 2, outer, 0)

    pltpu.sync_copy(acc, shared.at[s])
    plsc.subcore_barrier()

    stride = 1
    while stride < WPB:
        @pl.when(rank % (2 * stride) == 0)
        def _merge():
            pltpu.sync_copy(shared.at[s + stride], pacc)

            def mbody(j, carry):
                sl = pl.ds(j * L, L)
                acc[0, sl] = jnp.maximum(acc[0, sl], pacc[0, sl])
                acc[1, sl] = jnp.minimum(acc[1, sl], pacc[1, sl])
                acc[2, sl] = acc[2, sl] + pacc[2, sl]
                acc[3, sl] = acc[3, sl] + pacc[3, sl]
                return carry

            lax.fori_loop(0, NJ, mbody, 0)
            pltpu.sync_copy(acc, shared.at[s])

        plsc.subcore_barrier()
        stride *= 2

    @pl.when(rank == 0)
    def _finalize():
        def fbody(j, carry):
            sl = pl.ds(j * L, L)
            sm = acc[2, sl]
            mean = sm * jnp.float32(1.0 / N)
            var = jnp.maximum(
                (acc[3, sl] - sm * mean) * jnp.float32(1.0 / (N - 1)), 0.0)
            acc[2, sl] = mean
            acc[3, sl] = _sqrt_sc(var)
            return carry

        lax.fori_loop(0, NJ, fbody, 0)
        pltpu.sync_copy(acc, out_hbm.at[b])


def _pool_tc_body(x_ref, o_ref):
    xb = x_ref[0]
    mx = jnp.max(xb, axis=0)
    mn = jnp.min(xb, axis=0)
    sm = jnp.sum(xb, axis=0)
    ss = jnp.sum(xb * xb, axis=0)
    mean = sm * jnp.float32(1.0 / N)
    var = jnp.maximum((ss - sm * mean) * jnp.float32(1.0 / (N - 1)), 0.0)
    o_ref[0] = jnp.stack([mx, mn, mean, jnp.sqrt(var)], axis=0)


_pool_tc = pl.pallas_call(
    _pool_tc_body,
    grid=(B - BS,),
    in_specs=[pl.BlockSpec((1, N, D), lambda i: (i + BS, 0, 0))],
    out_specs=pl.BlockSpec((1, 4, D), lambda i: (i, 0, 0)),
    out_shape=jax.ShapeDtypeStruct((B - BS, 4, D), jnp.float32),
)


def kernel(x):
    return jnp.concatenate(
        [_pool_sc(x.reshape(B, N * D)), _pool_tc(x)], axis=0)


# speedup vs baseline: 1.8195x; 1.8195x over previous
import functools

import jax
import jax.numpy as jnp
from jax import lax
from jax.experimental import pallas as pl
from jax.experimental.pallas import tpu as pltpu
from jax.experimental.pallas import tpu_sc as plsc

B, N, D = 16, 4096, 256
NC, NS, L = 2, 16, 16
NJ = D // L
BS = 2
BPC = BS // NC
WPB = NS // BPC
RW = N // WPB
CH = 128
NCHUNK = RW // CH
RSTEP = 8

_mesh = plsc.VectorSubcoreMesh(core_axis_name="c", subcore_axis_name="s")


def _tree(op, xs):
    while len(xs) > 1:
        xs = [op(xs[i], xs[i + 1]) for i in range(0, len(xs) - 1, 2)] + (
            [xs[-1]] if len(xs) % 2 else [])
    return xs[0]


def _sqrt_sc(v):
    s = jnp.full((L,), 1.0, jnp.float32)
    for t in range(1, 17):
        s = jnp.where(v >= jnp.float32(4.0 ** t), s * 2.0, s)
        s = jnp.where(v < jnp.float32(4.0 ** (1 - t)), s * 0.5, s)
    for _ in range(4):
        s = jnp.float32(0.5) * (s + v / s)
    return jnp.where(v > 0.0, s, 0.0)


@functools.partial(
    pl.kernel,
    mesh=_mesh,
    out_type=jax.ShapeDtypeStruct((BS, 4, D), jnp.float32),
    scratch_types=[
        pltpu.VMEM((CH, D), jnp.float32),
        pltpu.VMEM((CH, D), jnp.float32),
        pltpu.VMEM((4, D), jnp.float32),
        pltpu.VMEM((4, D), jnp.float32),
        pltpu.VMEM_SHARED((NS, 4, D), jnp.float32),
        pltpu.SemaphoreType.DMA,
        pltpu.SemaphoreType.DMA,
    ],
)
def _pool_sc(x_hbm, out_hbm, buf0, buf1, acc, pacc, shared, sem0, sem1):
    c = lax.axis_index("c")
    s = lax.axis_index("s")
    b = c * BPC + s // WPB
    rank = s % WPB
    r0 = rank * RW

    def ibody(j, carry):
        sl = pl.ds(j * L, L)
        acc[0, sl] = jnp.full((L,), -jnp.inf, jnp.float32)
        acc[1, sl] = jnp.full((L,), jnp.inf, jnp.float32)
        acc[2, sl] = jnp.zeros((L,), jnp.float32)
        acc[3, sl] = jnp.zeros((L,), jnp.float32)
        return carry

    lax.fori_loop(0, NJ, ibody, 0)

    def copy(k, buf, sem):
        return pltpu.make_async_copy(
            x_hbm.at[b, pl.ds(r0 + k * CH, CH), :], buf, sem)

    copy(0, buf0, sem0).start()
    copy(1, buf1, sem1).start()

    def process(buf):
        def jbody(j, carry):
            sl = pl.ds(j * L, L)

            @plsc.parallel_loop(
                0, CH, step=RSTEP,
                carry=(acc[0, sl], acc[1, sl], acc[2, sl], acc[3, sl]))
            def rbody(r, t):
                mx, mn, sm, ss = t
                v = [buf[r + u, sl] for u in range(RSTEP)]
                mx = jnp.maximum(mx, _tree(jnp.maximum, v))
                mn = jnp.minimum(mn, _tree(jnp.minimum, v))
                sm = sm + _tree(lax.add, v)
                ss = ss + _tree(lax.add, [vi * vi for vi in v])
                return (mx, mn, sm, ss)

            mx, mn, sm, ss = rbody
            acc[0, sl] = mx
            acc[1, sl] = mn
            acc[2, sl] = sm
            acc[3, sl] = ss
            return carry

        lax.fori_loop(0, NJ, jbody, 0)

    def outer(i, carry):
        kk = i * 2
        copy(kk, buf0, sem0).wait()
        process(buf0)

        @pl.when(kk + 2 < NCHUNK)
        def _():
            copy(kk + 2, buf0, sem0).start()

        copy(kk + 1, buf1, sem1).wait()
        process(buf1)

        @pl.when(kk + 3 < NCHUNK)
        def _():
            copy(kk + 3, buf1, sem1).start()

        return carry

    lax.fori_loop(0, NCHUNK // 2, outer, 0)

    pltpu.sync_copy(acc, shared.at[s])
    plsc.subcore_barrier()

    stride = 1
    while stride < WPB:
        @pl.when(rank % (2 * stride) == 0)
        def _merge():
            pltpu.sync_copy(shared.at[s + stride], pacc)

            def mbody(j, carry):
                sl = pl.ds(j * L, L)
                acc[0, sl] = jnp.maximum(acc[0, sl], pacc[0, sl])
                acc[1, sl] = jnp.minimum(acc[1, sl], pacc[1, sl])
                acc[2, sl] = acc[2, sl] + pacc[2, sl]
                acc[3, sl] = acc[3, sl] + pacc[3, sl]
                return carry

            lax.fori_loop(0, NJ, mbody, 0)
            pltpu.sync_copy(acc, shared.at[s])

        plsc.subcore_barrier()
        stride *= 2

    @pl.when(rank == 0)
    def _finalize():
        def fbody(j, carry):
            sl = pl.ds(j * L, L)
            sm = acc[2, sl]
            mean = sm * jnp.float32(1.0 / N)
            var = jnp.maximum(
                (acc[3, sl] - sm * mean) * jnp.float32(1.0 / (N - 1)), 0.0)
            acc[2, sl] = mean
            acc[3, sl] = _sqrt_sc(var)
            return carry

        lax.fori_loop(0, NJ, fbody, 0)
        pltpu.sync_copy(acc, out_hbm.at[b])


DF = 128


def _pool_tc_body(x_ref, o_ref):
    xb = x_ref[0]
    mx = jnp.max(xb, axis=0)
    mn = jnp.min(xb, axis=0)
    sm = jnp.sum(xb, axis=0)
    ss = jnp.sum(xb * xb, axis=0)
    mean = sm * jnp.float32(1.0 / N)
    var = jnp.maximum((ss - sm * mean) * jnp.float32(1.0 / (N - 1)), 0.0)
    o_ref[0] = jnp.stack([mx, mn, mean, jnp.sqrt(var)], axis=0)


_pool_tc = pl.pallas_call(
    _pool_tc_body,
    grid=(B - BS, D // DF),
    in_specs=[pl.BlockSpec((1, N, DF), lambda i, f: (i + BS, 0, f))],
    out_specs=pl.BlockSpec((1, 4, DF), lambda i, f: (i, 0, f)),
    out_shape=jax.ShapeDtypeStruct((B - BS, 4, D), jnp.float32),
)


def kernel(x):
    return jnp.concatenate([_pool_sc(x), _pool_tc(x)], axis=0)


# speedup vs baseline: 2.2242x; 1.2224x over previous
import functools

import jax
import jax.numpy as jnp
from jax import lax
from jax.experimental import pallas as pl
from jax.experimental.pallas import tpu as pltpu
from jax.experimental.pallas import tpu_sc as plsc

B, N, D = 16, 4096, 256
NC, NS, L = 2, 16, 16
NJ = D // L
BS = 4
BPC = BS // NC
WPB = NS // BPC
RW = N // WPB
CH = 128
NCHUNK = RW // CH
RSTEP = 8

_mesh = plsc.VectorSubcoreMesh(core_axis_name="c", subcore_axis_name="s")


def _tree(op, xs):
    while len(xs) > 1:
        xs = [op(xs[i], xs[i + 1]) for i in range(0, len(xs) - 1, 2)] + (
            [xs[-1]] if len(xs) % 2 else [])
    return xs[0]


def _sqrt_sc(v):
    s = jnp.full((L,), 1.0, jnp.float32)
    for t in range(1, 17):
        s = jnp.where(v >= jnp.float32(4.0 ** t), s * 2.0, s)
        s = jnp.where(v < jnp.float32(4.0 ** (1 - t)), s * 0.5, s)
    for _ in range(4):
        s = jnp.float32(0.5) * (s + v / s)
    return jnp.where(v > 0.0, s, 0.0)


@functools.partial(
    pl.kernel,
    mesh=_mesh,
    out_type=jax.ShapeDtypeStruct((BS, 4, D), jnp.float32),
    scratch_types=[
        pltpu.VMEM((CH, D), jnp.float32),
        pltpu.VMEM((CH, D), jnp.float32),
        pltpu.VMEM((4, D), jnp.float32),
        pltpu.VMEM((4, D), jnp.float32),
        pltpu.VMEM_SHARED((NS, 4, D), jnp.float32),
        pltpu.SemaphoreType.DMA,
        pltpu.SemaphoreType.DMA,
    ],
)
def _pool_sc(x_hbm, out_hbm, buf0, buf1, acc, pacc, shared, sem0, sem1):
    c = lax.axis_index("c")
    s = lax.axis_index("s")
    b = c * BPC + s // WPB
    rank = s % WPB
    r0 = rank * RW

    def ibody(j, carry):
        sl = pl.ds(j * L, L)
        acc[0, sl] = jnp.full((L,), -jnp.inf, jnp.float32)
        acc[1, sl] = jnp.full((L,), jnp.inf, jnp.float32)
        acc[2, sl] = jnp.zeros((L,), jnp.float32)
        acc[3, sl] = jnp.zeros((L,), jnp.float32)
        return carry

    lax.fori_loop(0, NJ, ibody, 0)

    def copy(k, buf, sem):
        return pltpu.make_async_copy(
            x_hbm.at[b, pl.ds(r0 + k * CH, CH), :], buf, sem)

    copy(0, buf0, sem0).start()
    copy(1, buf1, sem1).start()

    def process(buf):
        def jbody(j, carry):
            sl = pl.ds(j * L, L)

            @plsc.parallel_loop(
                0, CH, step=RSTEP,
                carry=(acc[0, sl], acc[1, sl], acc[2, sl], acc[3, sl]))
            def rbody(r, t):
                mx, mn, sm, ss = t
                v = [buf[r + u, sl] for u in range(RSTEP)]
                mx = jnp.maximum(mx, _tree(jnp.maximum, v))
                mn = jnp.minimum(mn, _tree(jnp.minimum, v))
                sm = sm + _tree(lax.add, v)
                ss = ss + _tree(lax.add, [vi * vi for vi in v])
                return (mx, mn, sm, ss)

            mx, mn, sm, ss = rbody
            acc[0, sl] = mx
            acc[1, sl] = mn
            acc[2, sl] = sm
            acc[3, sl] = ss
            return carry

        lax.fori_loop(0, NJ, jbody, 0)

    def outer(i, carry):
        kk = i * 2
        copy(kk, buf0, sem0).wait()
        process(buf0)

        @pl.when(kk + 2 < NCHUNK)
        def _():
            copy(kk + 2, buf0, sem0).start()

        copy(kk + 1, buf1, sem1).wait()
        process(buf1)

        @pl.when(kk + 3 < NCHUNK)
        def _():
            copy(kk + 3, buf1, sem1).start()

        return carry

    lax.fori_loop(0, NCHUNK // 2, outer, 0)

    pltpu.sync_copy(acc, shared.at[s])
    plsc.subcore_barrier()

    stride = 1
    while stride < WPB:
        @pl.when(rank % (2 * stride) == 0)
        def _merge():
            pltpu.sync_copy(shared.at[s + stride], pacc)

            def mbody(j, carry):
                sl = pl.ds(j * L, L)
                acc[0, sl] = jnp.maximum(acc[0, sl], pacc[0, sl])
                acc[1, sl] = jnp.minimum(acc[1, sl], pacc[1, sl])
                acc[2, sl] = acc[2, sl] + pacc[2, sl]
                acc[3, sl] = acc[3, sl] + pacc[3, sl]
                return carry

            lax.fori_loop(0, NJ, mbody, 0)
            pltpu.sync_copy(acc, shared.at[s])

        plsc.subcore_barrier()
        stride *= 2

    @pl.when(rank == 0)
    def _finalize():
        def fbody(j, carry):
            sl = pl.ds(j * L, L)
            sm = acc[2, sl]
            mean = sm * jnp.float32(1.0 / N)
            var = jnp.maximum(
                (acc[3, sl] - sm * mean) * jnp.float32(1.0 / (N - 1)), 0.0)
            acc[2, sl] = mean
            acc[3, sl] = _sqrt_sc(var)
            return carry

        lax.fori_loop(0, NJ, fbody, 0)
        pltpu.sync_copy(acc, out_hbm.at[b])


def _pool_tc_body(x_ref, o_ref):
    xb = x_ref[0]
    mx = jnp.max(xb, axis=0)
    mn = jnp.min(xb, axis=0)
    sm = jnp.sum(xb, axis=0)
    ss = jnp.sum(xb * xb, axis=0)
    mean = sm * jnp.float32(1.0 / N)
    var = jnp.maximum((ss - sm * mean) * jnp.float32(1.0 / (N - 1)), 0.0)
    o_ref[0] = jnp.stack([mx, mn, mean, jnp.sqrt(var)], axis=0)


_pool_tc = pl.pallas_call(
    _pool_tc_body,
    grid=(B - BS,),
    in_specs=[pl.BlockSpec((1, N, D), lambda i: (i + BS, 0, 0))],
    out_specs=pl.BlockSpec((1, 4, D), lambda i: (i, 0, 0)),
    out_shape=jax.ShapeDtypeStruct((B - BS, 4, D), jnp.float32),
)


def kernel(x):
    return jnp.concatenate([_pool_sc(x), _pool_tc(x)], axis=0)
